# Initial kernel scaffold; baseline (speedup 1.0000x reference)
import jax
import jax.numpy as jnp
from jax.experimental import pallas as pl


def kernel(x, table):
    raise NotImplementedError("write your pallas kernel here")


# speedup vs baseline: 2.9336x; 2.9336x over previous
import functools

import jax
import jax.numpy as jnp
from jax import lax
from jax.experimental import pallas as pl
from jax.experimental.pallas import tpu as pltpu
from jax.experimental.pallas import tpu_sc as plsc


def _make_emb_kernel(B, D, NW, n_ch, CH):
    b_per_w = B // NW
    mesh = plsc.VectorSubcoreMesh(core_axis_name="c", subcore_axis_name="s")

    @functools.partial(
        pl.kernel,
        mesh=mesh,
        out_type=jax.ShapeDtypeStruct((B, D), jnp.float32),
        scratch_types=[
            pltpu.VMEM((n_ch, CH), jnp.int32),
            pltpu.VMEM((CH, D), jnp.float32),
            pltpu.SemaphoreType.DMA,
        ],
    )
    def emb(table_hbm, idx_hbm, out_hbm, idx_v, rows_v, sem):
        wid = lax.axis_index("s") * 2 + lax.axis_index("c")
        base = wid * b_per_w
        pltpu.sync_copy(idx_hbm.at[wid], idx_v)

        def body(j, carry):
            pltpu.async_copy(table_hbm.at[idx_v.at[j]], rows_v, sem).wait()
            pltpu.sync_copy(rows_v, out_hbm.at[pl.ds(base + j * CH, CH)])
            return carry

        lax.fori_loop(0, n_ch, body, 0)

    return emb


def kernel(x, table):
    B0, B1 = x.shape
    B = B0 * B1
    D = table.shape[1]
    info = plsc.get_sparse_core_info()
    NW = info.num_cores * info.num_subcores
    CH = 128
    b_per_w = B // NW
    n_ch = b_per_w // CH
    idx = x.reshape(NW, n_ch, CH).astype(jnp.int32)
    out = _make_emb_kernel(B, D, NW, n_ch, CH)(table, idx)
    return out.reshape(B0, B1, D)


# speedup vs baseline: 3.2829x; 1.1191x over previous
import functools

import jax
import jax.numpy as jnp
from jax import lax
from jax.experimental import pallas as pl
from jax.experimental.pallas import tpu as pltpu
from jax.experimental.pallas import tpu_sc as plsc


def _make_emb_kernel(B, D, NW, n_ch, CH, NB):
    b_per_w = B // NW
    n_t = n_ch // NB
    mesh = plsc.VectorSubcoreMesh(core_axis_name="c", subcore_axis_name="s")

    scratch = [pltpu.VMEM((n_ch, CH), jnp.int32)]
    scratch += [pltpu.VMEM((CH, D), jnp.float32) for _ in range(NB)]
    scratch += [pltpu.SemaphoreType.DMA for _ in range(2 * NB)]

    @functools.partial(
        pl.kernel,
        mesh=mesh,
        out_type=jax.ShapeDtypeStruct((B, D), jnp.float32),
        scratch_types=scratch,
    )
    def emb(table_hbm, idx_hbm, out_hbm, idx_v, *rest):
        bufs = rest[:NB]
        gsem = rest[NB:2 * NB]
        osem = rest[2 * NB:]
        wid = lax.axis_index("s") * 2 + lax.axis_index("c")
        base = wid * b_per_w
        pltpu.sync_copy(idx_hbm.at[wid], idx_v)

        def gather_start(j, b):
            pltpu.async_copy(table_hbm.at[idx_v.at[j]], bufs[b], gsem[b])

        def gather_wait(b):
            pltpu.make_async_copy(
                table_hbm.at[idx_v.at[0]], bufs[b], gsem[b]).wait()

        def write_start(j, b):
            pltpu.async_copy(
                bufs[b], out_hbm.at[pl.ds(base + j * CH, CH)], osem[b])

        def write_wait(b):
            pltpu.make_async_copy(
                bufs[b], out_hbm.at[pl.ds(base, CH)], osem[b]).wait()

        for b in range(NB):
            gather_start(b, b)

        def body(t, carry):
            for b in range(NB):
                gather_wait(b)
                write_start(t * NB + b, b)
            for b in range(NB):
                write_wait(b)
                gather_start((t + 1) * NB + b, b)
            return carry

        lax.fori_loop(0, n_t - 1, body, 0)

        for b in range(NB):
            gather_wait(b)
            write_start((n_t - 1) * NB + b, b)
        for b in range(NB):
            write_wait(b)

    return emb


def kernel(x, table):
    B0, B1 = x.shape
    B = B0 * B1
    D = table.shape[1]
    info = plsc.get_sparse_core_info()
    NW = info.num_cores * info.num_subcores
    CH = 128
    NB = 5
    b_per_w = B // NW
    n_ch = b_per_w // CH
    idx = x.reshape(NW, n_ch, CH).astype(jnp.int32)
    out = _make_emb_kernel(B, D, NW, n_ch, CH, NB)(table, idx)
    return out.reshape(B0, B1, D)


# speedup vs baseline: 5.8343x; 1.7772x over previous
import functools

import jax
import jax.numpy as jnp
from jax import lax
from jax.experimental import pallas as pl
from jax.experimental.pallas import tpu as pltpu
from jax.experimental.pallas import tpu_sc as plsc


def _make_emb_kernel(NBATCH, SEQ, D, NW, NB):
    be_w = NBATCH // NW
    n_ch = be_w // 2
    CHI = 2 * SEQ
    n_t = n_ch // NB
    mesh = plsc.VectorSubcoreMesh(core_axis_name="c", subcore_axis_name="s")

    scratch = [pltpu.VMEM((n_ch, 128), jnp.int32)]
    scratch += [pltpu.VMEM((CHI, D), jnp.float32) for _ in range(NB)]
    scratch += [pltpu.SemaphoreType.DMA for _ in range(2 * NB)]

    @functools.partial(
        pl.kernel,
        mesh=mesh,
        out_type=jax.ShapeDtypeStruct((NBATCH, SEQ, D), jnp.float32),
        scratch_types=scratch,
    )
    def emb(table_hbm, idx_hbm, out_hbm, idx_v, *rest):
        bufs = rest[:NB]
        gsem = rest[NB:2 * NB]
        osem = rest[2 * NB:]
        wid = lax.axis_index("s") * 2 + lax.axis_index("c")
        be0 = wid * be_w
        pltpu.sync_copy(idx_hbm.at[wid], idx_v)

        def gather_start(j, b):
            pltpu.async_copy(
                table_hbm.at[idx_v.at[j, pl.ds(0, CHI)]], bufs[b], gsem[b])

        def gather_wait(b):
            pltpu.make_async_copy(
                table_hbm.at[idx_v.at[0, pl.ds(0, CHI)]], bufs[b],
                gsem[b]).wait()

        def write_start(j, b):
            be = be0 + 2 * j
            pltpu.async_copy(
                bufs[b].at[pl.ds(0, SEQ)], out_hbm.at[be], osem[b])
            pltpu.async_copy(
                bufs[b].at[pl.ds(SEQ, SEQ)], out_hbm.at[be + 1], osem[b])

        def write_wait(b):
            pltpu.make_async_copy(
                bufs[b].at[pl.ds(0, SEQ)], out_hbm.at[0], osem[b]).wait()
            pltpu.make_async_copy(
                bufs[b].at[pl.ds(SEQ, SEQ)], out_hbm.at[0], osem[b]).wait()

        for b in range(NB):
            gather_start(b, b)

        def body(t, carry):
            for b in range(NB):
                gather_wait(b)
                write_start(t * NB + b, b)
            for b in range(NB):
                write_wait(b)
                gather_start((t + 1) * NB + b, b)
            return carry

        lax.fori_loop(0, n_t - 1, body, 0)

        for b in range(NB):
            gather_wait(b)
            write_start((n_t - 1) * NB + b, b)
        for b in range(NB):
            write_wait(b)

    return emb


def kernel(x, table):
    NBATCH, SEQ = x.shape
    D = table.shape[1]
    info = plsc.get_sparse_core_info()
    NW = info.num_cores * info.num_subcores
    NB = 4
    be_w = NBATCH // NW
    n_ch = be_w // 2
    idx = x.reshape(NW, n_ch, 2 * SEQ).astype(jnp.int32)
    idx = jnp.pad(idx, ((0, 0), (0, 0), (0, 128 - 2 * SEQ)))
    out = _make_emb_kernel(NBATCH, SEQ, D, NW, NB)(table, idx)
    return out


# speedup vs baseline: 10.0579x; 1.7239x over previous
import functools

import jax
import jax.numpy as jnp
from jax import lax
from jax.experimental import pallas as pl
from jax.experimental.pallas import tpu as pltpu
from jax.experimental.pallas import tpu_sc as plsc


def _make_emb_kernel(B, D, NW, n_ch, CH, NB):
    b_per_w = B // NW
    n_t = n_ch // NB
    mesh = plsc.VectorSubcoreMesh(core_axis_name="c", subcore_axis_name="s")

    scratch = [pltpu.VMEM((n_ch, CH), jnp.int32)]
    scratch += [pltpu.VMEM((CH, D), jnp.float32) for _ in range(NB)]
    scratch += [pltpu.SemaphoreType.DMA for _ in range(2 * NB)]

    @functools.partial(
        pl.kernel,
        mesh=mesh,
        out_type=jax.ShapeDtypeStruct((B, D), jnp.float32),
        scratch_types=scratch,
    )
    def emb(table_hbm, idx_hbm, out_hbm, idx_v, *rest):
        bufs = rest[:NB]
        gsem = rest[NB:2 * NB]
        osem = rest[2 * NB:]
        wid = lax.axis_index("s") * 2 + lax.axis_index("c")
        base = wid * b_per_w
        pltpu.sync_copy(idx_hbm.at[wid], idx_v)

        def gather_start(j, b):
            pltpu.async_copy(table_hbm.at[idx_v.at[j]], bufs[b], gsem[b])

        def gather_wait(b):
            pltpu.make_async_copy(
                table_hbm.at[idx_v.at[0]], bufs[b], gsem[b]).wait()

        def write_start(j, b):
            pltpu.async_copy(
                bufs[b], out_hbm.at[pl.ds(base + j * CH, CH)], osem[b])

        def write_wait(b):
            pltpu.make_async_copy(
                bufs[b], out_hbm.at[pl.ds(base, CH)], osem[b]).wait()

        for b in range(NB):
            gather_start(b, b)

        def body(t, carry):
            for b in range(NB):
                gather_wait(b)
                write_start(t * NB + b, b)
            for b in range(NB):
                write_wait(b)
                gather_start((t + 1) * NB + b, b)
            return carry

        lax.fori_loop(0, n_t - 1, body, 0)

        for b in range(NB):
            gather_wait(b)
            write_start((n_t - 1) * NB + b, b)
        for b in range(NB):
            write_wait(b)

    return emb


def kernel(x, table):
    B0, B1 = x.shape
    B = B0 * B1
    D = table.shape[1]
    info = plsc.get_sparse_core_info()
    NW = info.num_cores * info.num_subcores
    CH = 128
    NB = 5
    b_per_w = B // NW
    n_ch = b_per_w // CH
    idx = x.T.reshape(NW, n_ch, CH).astype(jnp.int32)
    out = _make_emb_kernel(B, D, NW, n_ch, CH, NB)(table, idx)
    return out.reshape(B1, B0, D).transpose(1, 0, 2)
